# Initial kernel scaffold; baseline (speedup 1.0000x reference)
import jax
import jax.numpy as jnp
from jax.experimental import pallas as pl


def kernel(cell_embed, original_node, map, num_cell, costs, init_w, Wh, bh, Wv, bv, Wq, bq, Wr, br, v_ptr):
    raise NotImplementedError("write your pallas kernel here")


# speedup vs baseline: 88.1238x; 88.1238x over previous
import jax
import jax.numpy as jnp
from jax import lax
from jax.experimental import pallas as pl

C_CONST = 10.0


def _dot_t(x, w):
    return lax.dot_general(x, w, (((1,), (1,)), ((), ())),
                           preferred_element_type=jnp.float32)


def _decode_tc(cce_b, cc0, cc1, cc2, cc3, costs2, G_t, init_w, Wh, bh, Wv,
               bv, Wq, bq, Wr, br, v_ptr, interpret=False):
    B, n4, NE = cce_b.shape
    item = G_t.shape[0]

    def body(cce_ref, cc0_ref, cc1_ref, cc2_ref, cc3_ref, costs_ref, G_ref,
             iw_ref, Wh_ref, bh_ref, Wv_ref, bv_ref, Wq_ref, bq_ref, Wr_ref,
             br_ref, vp_ref, lp_ref, rew_ref, act_ref):
        cce = cce_ref[...]
        Wv_ = Wv_ref[...]
        Wq_ = Wq_ref[...]
        bq_ = bq_ref[...]
        vp = vp_ref[...]
        cc0 = cc0_ref[...]
        cc1 = cc1_ref[...]
        cc2 = cc2_ref[...]
        cc3 = cc3_ref[...]
        costs = costs_ref[...]

        h_mean = jnp.mean(cce, axis=1)
        h_bar = _dot_t(h_mean, Wh_ref[...]) + bh_ref[...]
        ref_proj = (_dot_t(cce.reshape(B * n4, NE), Wr_ref[...])
                    + br_ref[...]).reshape(B, n4, NE)
        hrest0 = _dot_t(iw_ref[...], Wv_) + bv_ref[...]
        query0 = h_bar + hrest0
        cce_row0 = cce[:, 0, :]

        it_n4 = lax.broadcasted_iota(jnp.int32, (B, n4), 1)
        it_ne = lax.broadcasted_iota(jnp.int32, (B, NE), 1)
        it_item = lax.broadcasted_iota(jnp.int32, (B, item), 1)

        def step(i, carry):
            (query, mask, aval, sptx, spty, cprev, lp, rew, acts) = carry
            q = _dot_t(query, Wq_) + bq_
            u = C_CONST * jnp.tanh(q[:, None, :] + ref_proj)
            s = lax.dot_general(u.reshape(B * n4, NE), vp,
                                (((1,), (0,)), ((), ()))).reshape(B, n4)
            s = jnp.where(mask == 1, -1e9, s)
            m = jnp.max(s, axis=1, keepdims=True)
            e = jnp.exp(s - m)
            p = e / jnp.sum(e, axis=1, keepdims=True)
            logits = jnp.log(p + 1e-20)
            z = logits + G_ref[i]
            zmax = jnp.max(z, axis=1, keepdims=True)
            idx = jnp.min(jnp.where(z == zmax, it_n4, n4 + 1), axis=1,
                          keepdims=True)
            oh = (it_n4 == idx).astype(jnp.float32)
            p_sel = jnp.sum(p * oh, axis=1, keepdims=True)
            lp = lp + jnp.log(p_sel + 1e-20)
            mask = jnp.where((it_n4 >> 2) == (idx >> 2), 1, mask)
            ex = jnp.sum(cc0 * oh, axis=1, keepdims=True)
            ey = jnp.sum(cc1 * oh, axis=1, keepdims=True)
            ext = jnp.sqrt((ex - sptx) ** 2 + (ey - spty) ** 2)
            ccur = jnp.sum(costs * oh, axis=1, keepdims=True)
            rew = rew + jnp.where(i > 0, ext + cprev + ccur, 0.0)
            sptx = jnp.sum(cc2 * oh, axis=1, keepdims=True)
            spty = jnp.sum(cc3 * oh, axis=1, keepdims=True)
            idxc = jnp.minimum(idx, NE - 1)
            oh_ne = (it_ne == idxc).astype(jnp.float32)
            bval = jnp.sum(cce_row0 * oh_ne, axis=1, keepdims=True)
            aval = jnp.where(i == 0, bval, aval)
            concat = jnp.concatenate(
                [jnp.broadcast_to(aval, (B, NE)),
                 jnp.broadcast_to(bval, (B, NE))], axis=1)
            h_rest = _dot_t(concat, Wv_) + bv_ref[...]
            query = h_bar + h_rest
            acts = acts + idx * (it_item == i).astype(jnp.int32)
            return (query, mask, aval, sptx, spty, ccur, lp, rew, acts)

        zB1 = jnp.zeros((B, 1), jnp.float32)
        carry0 = (query0, jnp.zeros((B, n4), jnp.int32), zB1, zB1, zB1, zB1,
                  zB1, zB1, jnp.zeros((B, item), jnp.int32))
        (_, _, _, _, _, _, lp, rew, acts) = lax.fori_loop(0, item, step,
                                                          carry0)
        lp_ref[...] = lp
        rew_ref[...] = rew
        act_ref[...] = acts

    out_shape = (
        jax.ShapeDtypeStruct((B, 1), jnp.float32),
        jax.ShapeDtypeStruct((B, 1), jnp.float32),
        jax.ShapeDtypeStruct((B, item), jnp.int32),
    )
    lp, rew, acts = pl.pallas_call(body, out_shape=out_shape,
                                   interpret=interpret)(
        cce_b, cc0, cc1, cc2, cc3, costs2, G_t, init_w, Wh, bh, Wv, bv, Wq,
        bq, Wr, br, v_ptr)
    return lp[:, 0], rew[:, 0], acts


def _prep(cell_embed, original_node, num_cell, costs, init_w):
    B = num_cell.shape[0]
    item = cell_embed.shape[0] // (4 * B)
    n4 = 4 * item
    NE = cell_embed.shape[1]
    starts = jnp.concatenate(
        [jnp.zeros((1,), jnp.int32), (4 * num_cell[:-1]).astype(jnp.int32)])
    cce_b = jax.vmap(
        lambda s: lax.dynamic_slice(cell_embed, (s, 0), (n4, NE)))(starts)
    cc_b = jax.vmap(
        lambda s: lax.dynamic_slice(original_node, (s, 0), (n4, 4)))(starts)
    cc0, cc1, cc2, cc3 = (cc_b[:, :, k] for k in range(4))
    costs2 = costs.reshape(B, n4)
    base = jax.random.key(1234)
    steps = jnp.arange(B * item)
    keys = jax.vmap(lambda s: jax.random.fold_in(base, s))(steps)
    G = jax.vmap(lambda k: jax.random.gumbel(k, (n4,), jnp.float32))(keys)
    G_t = G.reshape(B, item, n4).transpose(1, 0, 2)
    return cce_b, cc0, cc1, cc2, cc3, costs2, G_t, init_w.reshape(1, -1)


def kernel(cell_embed, original_node, map, num_cell, costs, init_w, Wh, bh,
           Wv, bv, Wq, bq, Wr, br, v_ptr):
    cce_b, cc0, cc1, cc2, cc3, costs2, G_t, iw = _prep(
        cell_embed, original_node, num_cell, costs, init_w)
    lp, rew, acts = _decode_tc(
        cce_b, cc0, cc1, cc2, cc3, costs2, G_t, iw, Wh, bh.reshape(1, -1),
        Wv, bv.reshape(1, -1), Wq, bq.reshape(1, -1), Wr, br.reshape(1, -1),
        v_ptr.reshape(-1, 1))
    return lp, rew, acts
